# baseline (speedup 1.0000x reference)
import jax
import jax.numpy as jnp
from jax.experimental import pallas as pl
from jax.experimental.pallas import tpu as pltpu

N = 10000
IN_C = 512
HID = 256
OUT_C = 16
K = 10
ALPHA = 0.1

ADJ_SCALE = 16384.0
BR = 400
NB = N // BR
XBR = 1000
XNB = N // XBR
F8 = jnp.float8_e4m3fn


def _mlp_kernel(x_ref, w1_ref, b1_ref, w2_ref, b2_ref, out_ref):
    h = jnp.dot(x_ref[...], w1_ref[...], preferred_element_type=jnp.float32)
    h = jnp.maximum(h + b1_ref[...], 0.0)
    out_ref[...] = (
        jnp.dot(h, w2_ref[...], preferred_element_type=jnp.float32)
        + b2_ref[...]
    )


def _quant_step0_kernel(adj_ref, x2_ref, a8_ref, s1_ref, rc_ref, c_ref, s_ref):
    r = pl.program_id(0)

    @pl.when(r == 0)
    def _():
        c0 = jnp.mean(x2_ref[...], axis=0, keepdims=True)
        c_ref[...] = jnp.broadcast_to(c0, (8, OUT_C))
        s_ref[:, :OUT_C] = (x2_ref[...] - c0).astype(F8)
        s_ref[:, OUT_C:] = jnp.ones((N, OUT_C), F8)

    q = (adj_ref[...] * ADJ_SCALE).astype(F8)
    a8_ref[...] = q
    d = jnp.dot(q, s_ref[...], preferred_element_type=jnp.float32)
    c = c_ref[0:1, :]
    rc = (d[:, OUT_C:] * (1.0 / ADJ_SCALE)) * c
    rc_ref[...] = rc
    z1 = (
        ((1.0 - ALPHA) / ADJ_SCALE) * d[:, :OUT_C]
        + (1.0 - ALPHA) * rc
        + ALPHA * x2_ref[pl.ds(r * BR, BR), :]
    )
    s1_ref[...] = (z1 - c).astype(F8)


def _prop_step_kernel(a8_ref, x2_ref, sp_ref, rc_ref, c_ref, sn_ref):
    r = pl.program_id(0)
    d = jnp.dot(a8_ref[...], sp_ref[...], preferred_element_type=jnp.float32)
    y = (
        ((1.0 - ALPHA) / ADJ_SCALE) * d
        + (1.0 - ALPHA) * rc_ref[pl.ds(r * BR, BR), :]
        + ALPHA * x2_ref[pl.ds(r * BR, BR), :]
    )
    sn_ref[...] = (y - c_ref[0:1, :]).astype(F8)


def _prop_last_kernel(a8_ref, x2_ref, sp_ref, rc_ref, c_ref, out_ref):
    r = pl.program_id(0)
    d = jnp.dot(a8_ref[...], sp_ref[...], preferred_element_type=jnp.float32)
    out_ref[...] = (
        ((1.0 - ALPHA) / ADJ_SCALE) * d
        + (1.0 - ALPHA) * rc_ref[pl.ds(r * BR, BR), :]
        + ALPHA * x2_ref[pl.ds(r * BR, BR), :]
    )


def kernel(x, adj, W1, b1, W2, b2):
    b1r = b1.reshape(1, HID)
    b2r = b2.reshape(1, OUT_C)

    x2 = pl.pallas_call(
        _mlp_kernel,
        grid=(XNB,),
        in_specs=[
            pl.BlockSpec((XBR, IN_C), lambda i: (i, 0)),
            pl.BlockSpec((IN_C, HID), lambda i: (0, 0)),
            pl.BlockSpec((1, HID), lambda i: (0, 0)),
            pl.BlockSpec((HID, OUT_C), lambda i: (0, 0)),
            pl.BlockSpec((1, OUT_C), lambda i: (0, 0)),
        ],
        out_specs=pl.BlockSpec((XBR, OUT_C), lambda i: (i, 0)),
        out_shape=jax.ShapeDtypeStruct((N, OUT_C), jnp.float32),
    )(x, W1, b1r, W2, b2r)

    a8, s1, rc, c = pl.pallas_call(
        _quant_step0_kernel,
        grid=(NB,),
        in_specs=[
            pl.BlockSpec((BR, N), lambda r: (r, 0)),
            pl.BlockSpec((N, OUT_C), lambda r: (0, 0)),
        ],
        out_specs=[
            pl.BlockSpec((BR, N), lambda r: (r, 0)),
            pl.BlockSpec((BR, OUT_C), lambda r: (r, 0)),
            pl.BlockSpec((BR, OUT_C), lambda r: (r, 0)),
            pl.BlockSpec((8, OUT_C), lambda r: (0, 0)),
        ],
        out_shape=[
            jax.ShapeDtypeStruct((N, N), F8),
            jax.ShapeDtypeStruct((N, OUT_C), F8),
            jax.ShapeDtypeStruct((N, OUT_C), jnp.float32),
            jax.ShapeDtypeStruct((8, OUT_C), jnp.float32),
        ],
        scratch_shapes=[pltpu.VMEM((N, 2 * OUT_C), F8)],
    )(adj, x2)

    step_specs = dict(
        grid=(NB,),
        in_specs=[
            pl.BlockSpec((BR, N), lambda r: (r, 0)),
            pl.BlockSpec((N, OUT_C), lambda r: (0, 0)),
            pl.BlockSpec((N, OUT_C), lambda r: (0, 0)),
            pl.BlockSpec((N, OUT_C), lambda r: (0, 0)),
            pl.BlockSpec((8, OUT_C), lambda r: (0, 0)),
        ],
        out_specs=pl.BlockSpec((BR, OUT_C), lambda r: (r, 0)),
    )
    s = s1
    for _ in range(K - 2):
        s = pl.pallas_call(
            _prop_step_kernel,
            out_shape=jax.ShapeDtypeStruct((N, OUT_C), F8),
            **step_specs,
        )(a8, x2, s, rc, c)
    z = pl.pallas_call(
        _prop_last_kernel,
        out_shape=jax.ShapeDtypeStruct((N, OUT_C), jnp.float32),
        **step_specs,
    )(a8, x2, s, rc, c)
    return z


# speedup vs baseline: 1.2560x; 1.2560x over previous
import jax
import jax.numpy as jnp
from jax.experimental import pallas as pl
from jax.experimental.pallas import tpu as pltpu

N = 10000
IN_C = 512
HID = 256
OUT_C = 16
K = 10
ALPHA = 0.1

ADJ_SCALE = 16384.0
BR = 400
NB = N // BR
XBR = 1000
XNB = N // XBR
F8 = jnp.float8_e4m3fn


def _mlp_kernel(x_ref, w1_ref, b1_ref, w2_ref, b2_ref, out_ref):
    h = jnp.dot(x_ref[...], w1_ref[...], preferred_element_type=jnp.float32)
    h = jnp.maximum(h + b1_ref[...], 0.0)
    out_ref[...] = (
        jnp.dot(h, w2_ref[...], preferred_element_type=jnp.float32)
        + b2_ref[...]
    )


def _quant_step0_kernel(adj_ref, x2_ref, a8_ref, s1_ref, rc_ref, c_ref, s_ref):
    r = pl.program_id(0)

    @pl.when(r == 0)
    def _():
        c0 = jnp.mean(x2_ref[...], axis=0, keepdims=True)
        c_ref[...] = jnp.broadcast_to(c0, (8, OUT_C))
        s_ref[:, :OUT_C] = (x2_ref[...] - c0).astype(F8)
        s_ref[:, OUT_C:] = jnp.ones((N, OUT_C), F8)

    q = (adj_ref[...] * ADJ_SCALE).astype(F8)
    a8_ref[...] = q
    d = jnp.dot(q, s_ref[...], preferred_element_type=jnp.float32)
    c = c_ref[0:1, :]
    rc = (d[:, OUT_C:] * (1.0 / ADJ_SCALE)) * c
    rc_ref[...] = rc
    z1 = (
        ((1.0 - ALPHA) / ADJ_SCALE) * d[:, :OUT_C]
        + (1.0 - ALPHA) * rc
        + ALPHA * x2_ref[pl.ds(r * BR, BR), :]
    )
    s1_ref[...] = (z1 - c).astype(F8)


BR2 = 1000
NB2 = N // BR2


def _prop9_kernel(a8_ref, x2_ref, s1_ref, rc_ref, c_ref, out_ref, s_ref):
    j = pl.program_id(0)
    r = pl.program_id(1)

    @pl.when(jnp.logical_and(j == 0, r == 0))
    def _():
        s_ref[0] = s1_ref[...]

    d = jnp.dot(a8_ref[...], s_ref[j % 2], preferred_element_type=jnp.float32)
    y = (
        ((1.0 - ALPHA) / ADJ_SCALE) * d
        + (1.0 - ALPHA) * rc_ref[pl.ds(r * BR2, BR2), :]
        + ALPHA * x2_ref[pl.ds(r * BR2, BR2), :]
    )
    s_ref[(j + 1) % 2, pl.ds(r * BR2, BR2), :] = (y - c_ref[0:1, :]).astype(F8)

    @pl.when(j == K - 2)
    def _():
        out_ref[...] = y


def kernel(x, adj, W1, b1, W2, b2):
    b1r = b1.reshape(1, HID)
    b2r = b2.reshape(1, OUT_C)

    x2 = pl.pallas_call(
        _mlp_kernel,
        grid=(XNB,),
        in_specs=[
            pl.BlockSpec((XBR, IN_C), lambda i: (i, 0)),
            pl.BlockSpec((IN_C, HID), lambda i: (0, 0)),
            pl.BlockSpec((1, HID), lambda i: (0, 0)),
            pl.BlockSpec((HID, OUT_C), lambda i: (0, 0)),
            pl.BlockSpec((1, OUT_C), lambda i: (0, 0)),
        ],
        out_specs=pl.BlockSpec((XBR, OUT_C), lambda i: (i, 0)),
        out_shape=jax.ShapeDtypeStruct((N, OUT_C), jnp.float32),
    )(x, W1, b1r, W2, b2r)

    a8, s1, rc, c = pl.pallas_call(
        _quant_step0_kernel,
        grid=(NB,),
        in_specs=[
            pl.BlockSpec((BR, N), lambda r: (r, 0)),
            pl.BlockSpec((N, OUT_C), lambda r: (0, 0)),
        ],
        out_specs=[
            pl.BlockSpec((BR, N), lambda r: (r, 0)),
            pl.BlockSpec((BR, OUT_C), lambda r: (r, 0)),
            pl.BlockSpec((BR, OUT_C), lambda r: (r, 0)),
            pl.BlockSpec((8, OUT_C), lambda r: (0, 0)),
        ],
        out_shape=[
            jax.ShapeDtypeStruct((N, N), F8),
            jax.ShapeDtypeStruct((N, OUT_C), F8),
            jax.ShapeDtypeStruct((N, OUT_C), jnp.float32),
            jax.ShapeDtypeStruct((8, OUT_C), jnp.float32),
        ],
        scratch_shapes=[pltpu.VMEM((N, 2 * OUT_C), F8)],
    )(adj, x2)

    z = pl.pallas_call(
        _prop9_kernel,
        grid=(K - 1, NB2),
        in_specs=[
            pl.BlockSpec((BR2, N), lambda j, r: (r, 0)),
            pl.BlockSpec((N, OUT_C), lambda j, r: (0, 0)),
            pl.BlockSpec((N, OUT_C), lambda j, r: (0, 0)),
            pl.BlockSpec((N, OUT_C), lambda j, r: (0, 0)),
            pl.BlockSpec((8, OUT_C), lambda j, r: (0, 0)),
        ],
        out_specs=pl.BlockSpec(
            (BR2, OUT_C), lambda j, r: (jnp.where(j == K - 2, r, 0), 0)
        ),
        out_shape=jax.ShapeDtypeStruct((N, OUT_C), jnp.float32),
        scratch_shapes=[pltpu.VMEM((2, N, OUT_C), F8)],
    )(a8, x2, s1, rc, c)
    return z


# speedup vs baseline: 1.2755x; 1.0155x over previous
import jax
import jax.numpy as jnp
from jax.experimental import pallas as pl
from jax.experimental.pallas import tpu as pltpu

N = 10000
IN_C = 512
HID = 256
OUT_C = 16
K = 10
ALPHA = 0.1

ADJ_SCALE = 16384.0
BR = 400
NB = N // BR
BR2 = 1000
NB2 = N // BR2
XBR = 1000
XNB = N // XBR
CS = 8704
CR = N - CS
F8 = jnp.float8_e4m3fn


def _mlp_kernel(x_ref, w1_ref, b1_ref, w2_ref, b2_ref, out_ref):
    h = jnp.dot(x_ref[...], w1_ref[...], preferred_element_type=jnp.float32)
    h = jnp.maximum(h + b1_ref[...], 0.0)
    out_ref[...] = (
        jnp.dot(h, w2_ref[...], preferred_element_type=jnp.float32)
        + b2_ref[...]
    )


def _quant_step0_kernel(
    adj_ref, x2_ref, a8s_ref, a8r_ref, s1_ref, w_ref, c_ref, s_ref
):
    r = pl.program_id(0)

    @pl.when(r == 0)
    def _():
        c0 = jnp.mean(x2_ref[...], axis=0, keepdims=True)
        c_ref[...] = jnp.broadcast_to(c0, (8, OUT_C))
        s_ref[:, :OUT_C] = (x2_ref[...] - c0).astype(F8)
        s_ref[:, OUT_C:] = jnp.ones((N, OUT_C), F8)

    q = (adj_ref[...] * ADJ_SCALE).astype(F8)
    a8s_ref[...] = q[:, :CS]
    a8r_ref[...] = q[:, CS:]
    d = jnp.dot(q, s_ref[...], preferred_element_type=jnp.float32)
    c = c_ref[0:1, :]
    rc = (d[:, OUT_C:] * (1.0 / ADJ_SCALE)) * c
    w = (1.0 - ALPHA) * rc + ALPHA * x2_ref[pl.ds(r * BR, BR), :]
    w_ref[...] = w
    z1 = ((1.0 - ALPHA) / ADJ_SCALE) * d[:, :OUT_C] + w
    s1_ref[...] = (z1 - c).astype(F8)


def _prop9_kernel(a8s_ref, a8r_ref, x2w_ref, s1_ref, c_ref, out_ref, s_ref):
    j = pl.program_id(0)
    r = pl.program_id(1)

    @pl.when(jnp.logical_and(j == 0, r == 0))
    def _():
        s_ref[0] = s1_ref[...]

    d = jnp.dot(
        a8s_ref[...], s_ref[j % 2, :CS, :], preferred_element_type=jnp.float32
    ) + jnp.dot(
        a8r_ref[pl.ds(r * BR2, BR2), :],
        s_ref[j % 2, CS:, :],
        preferred_element_type=jnp.float32,
    )
    y = ((1.0 - ALPHA) / ADJ_SCALE) * d + x2w_ref[pl.ds(r * BR2, BR2), :]
    s_ref[(j + 1) % 2, pl.ds(r * BR2, BR2), :] = (y - c_ref[0:1, :]).astype(F8)

    @pl.when(j == K - 2)
    def _():
        out_ref[...] = y


def kernel(x, adj, W1, b1, W2, b2):
    b1r = b1.reshape(1, HID)
    b2r = b2.reshape(1, OUT_C)

    x2 = pl.pallas_call(
        _mlp_kernel,
        grid=(XNB,),
        in_specs=[
            pl.BlockSpec((XBR, IN_C), lambda i: (i, 0)),
            pl.BlockSpec((IN_C, HID), lambda i: (0, 0)),
            pl.BlockSpec((1, HID), lambda i: (0, 0)),
            pl.BlockSpec((HID, OUT_C), lambda i: (0, 0)),
            pl.BlockSpec((1, OUT_C), lambda i: (0, 0)),
        ],
        out_specs=pl.BlockSpec((XBR, OUT_C), lambda i: (i, 0)),
        out_shape=jax.ShapeDtypeStruct((N, OUT_C), jnp.float32),
    )(x, W1, b1r, W2, b2r)

    a8s, a8r, s1, w, c = pl.pallas_call(
        _quant_step0_kernel,
        grid=(NB,),
        in_specs=[
            pl.BlockSpec((BR, N), lambda r: (r, 0)),
            pl.BlockSpec((N, OUT_C), lambda r: (0, 0)),
        ],
        out_specs=[
            pl.BlockSpec((BR, CS), lambda r: (r, 0)),
            pl.BlockSpec((BR, CR), lambda r: (r, 0)),
            pl.BlockSpec((BR, OUT_C), lambda r: (r, 0)),
            pl.BlockSpec((BR, OUT_C), lambda r: (r, 0)),
            pl.BlockSpec((8, OUT_C), lambda r: (0, 0)),
        ],
        out_shape=[
            jax.ShapeDtypeStruct((N, CS), F8),
            jax.ShapeDtypeStruct((N, CR), F8),
            jax.ShapeDtypeStruct((N, OUT_C), F8),
            jax.ShapeDtypeStruct((N, OUT_C), jnp.float32),
            jax.ShapeDtypeStruct((8, OUT_C), jnp.float32),
        ],
        scratch_shapes=[pltpu.VMEM((N, 2 * OUT_C), F8)],
    )(adj, x2)

    z = pl.pallas_call(
        _prop9_kernel,
        grid=(K - 1, NB2),
        in_specs=[
            pl.BlockSpec((BR2, CS), lambda j, r: (r, 0)),
            pl.BlockSpec((N, CR), lambda j, r: (0, 0)),
            pl.BlockSpec((N, OUT_C), lambda j, r: (0, 0)),
            pl.BlockSpec((N, OUT_C), lambda j, r: (0, 0)),
            pl.BlockSpec((8, OUT_C), lambda j, r: (0, 0)),
        ],
        out_specs=pl.BlockSpec(
            (BR2, OUT_C), lambda j, r: (jnp.where(j == K - 2, r, 0), 0)
        ),
        out_shape=jax.ShapeDtypeStruct((N, OUT_C), jnp.float32),
        scratch_shapes=[pltpu.VMEM((2, N, OUT_C), F8)],
    )(a8s, a8r, w, s1, c)
    return z


# speedup vs baseline: 1.2789x; 1.0027x over previous
import jax
import jax.numpy as jnp
from jax.experimental import pallas as pl
from jax.experimental.pallas import tpu as pltpu

N = 10000
IN_C = 512
HID = 256
OUT_C = 16
K = 10
ALPHA = 0.1

ADJ_SCALE = 16384.0
BR = 400
NB = N // BR
BR2 = 1000
NB2 = N // BR2
XBR = 1000
XNB = N // XBR
CS = 8448
CR = N - CS
F8 = jnp.float8_e4m3fn


def _mlp_kernel(x_ref, w1_ref, b1_ref, w2_ref, b2_ref, out_ref):
    h = jnp.dot(x_ref[...], w1_ref[...], preferred_element_type=jnp.float32)
    h = jnp.maximum(h + b1_ref[...], 0.0)
    out_ref[...] = (
        jnp.dot(h, w2_ref[...], preferred_element_type=jnp.float32)
        + b2_ref[...]
    )


def _quant_step0_kernel(
    adj_ref, x2_ref, a8s_ref, a8r_ref, s1_ref, w_ref, c_ref, s_ref
):
    r = pl.program_id(0)

    @pl.when(r == 0)
    def _():
        c0 = jnp.mean(x2_ref[...], axis=0, keepdims=True)
        c_ref[...] = jnp.broadcast_to(c0, (8, OUT_C))
        s_ref[:, :OUT_C] = (x2_ref[...] - c0).astype(F8)
        s_ref[:, OUT_C:] = jnp.ones((N, OUT_C), F8)

    q = (adj_ref[...] * ADJ_SCALE).astype(F8)
    a8s_ref[...] = q[:, :CS]
    a8r_ref[...] = q[:, CS:]
    d = jnp.dot(q, s_ref[...], preferred_element_type=jnp.float32)
    c = c_ref[0:1, :]
    rc = (d[:, OUT_C:] * (1.0 / ADJ_SCALE)) * c
    w = (1.0 - ALPHA) * rc + ALPHA * x2_ref[pl.ds(r * BR, BR), :]
    w_ref[...] = w
    z1 = ((1.0 - ALPHA) / ADJ_SCALE) * d[:, :OUT_C] + w
    s1_ref[...] = (z1 - c).astype(F8)


def _prop9_kernel(a8s_ref, a8r_ref, x2w_ref, s1_ref, c_ref, out_ref, s_ref, sem):
    j = pl.program_id(0)
    r = pl.program_id(1)

    @pl.when(jnp.logical_and(j == 0, r == 0))
    def _():
        cp = pltpu.make_async_copy(s1_ref, s_ref.at[0], sem)
        cp.start()
        cp.wait()

    d = jnp.dot(
        a8s_ref[...], s_ref[j % 2, :CS, :], preferred_element_type=jnp.float32
    ) + jnp.dot(
        a8r_ref[pl.ds(r * BR2, BR2), :],
        s_ref[j % 2, CS:, :],
        preferred_element_type=jnp.float32,
    )
    y = ((1.0 - ALPHA) / ADJ_SCALE) * d + x2w_ref[pl.ds(r * BR2, BR2), :]
    s_ref[(j + 1) % 2, pl.ds(r * BR2, BR2), :] = (y - c_ref[0:1, :]).astype(F8)

    @pl.when(j == K - 2)
    def _():
        out_ref[...] = y


def kernel(x, adj, W1, b1, W2, b2):
    b1r = b1.reshape(1, HID)
    b2r = b2.reshape(1, OUT_C)

    x2 = pl.pallas_call(
        _mlp_kernel,
        grid=(XNB,),
        in_specs=[
            pl.BlockSpec((XBR, IN_C), lambda i: (i, 0)),
            pl.BlockSpec((IN_C, HID), lambda i: (0, 0)),
            pl.BlockSpec((1, HID), lambda i: (0, 0)),
            pl.BlockSpec((HID, OUT_C), lambda i: (0, 0)),
            pl.BlockSpec((1, OUT_C), lambda i: (0, 0)),
        ],
        out_specs=pl.BlockSpec((XBR, OUT_C), lambda i: (i, 0)),
        out_shape=jax.ShapeDtypeStruct((N, OUT_C), jnp.float32),
    )(x, W1, b1r, W2, b2r)

    a8s, a8r, s1, w, c = pl.pallas_call(
        _quant_step0_kernel,
        grid=(NB,),
        in_specs=[
            pl.BlockSpec((BR, N), lambda r: (r, 0)),
            pl.BlockSpec((N, OUT_C), lambda r: (0, 0)),
        ],
        out_specs=[
            pl.BlockSpec((BR, CS), lambda r: (r, 0)),
            pl.BlockSpec((BR, CR), lambda r: (r, 0)),
            pl.BlockSpec((BR, OUT_C), lambda r: (r, 0)),
            pl.BlockSpec((BR, OUT_C), lambda r: (r, 0)),
            pl.BlockSpec((8, OUT_C), lambda r: (0, 0)),
        ],
        out_shape=[
            jax.ShapeDtypeStruct((N, CS), F8),
            jax.ShapeDtypeStruct((N, CR), F8),
            jax.ShapeDtypeStruct((N, OUT_C), F8),
            jax.ShapeDtypeStruct((N, OUT_C), jnp.float32),
            jax.ShapeDtypeStruct((8, OUT_C), jnp.float32),
        ],
        scratch_shapes=[pltpu.VMEM((N, 2 * OUT_C), F8)],
    )(adj, x2)

    z = pl.pallas_call(
        _prop9_kernel,
        grid=(K - 1, NB2),
        in_specs=[
            pl.BlockSpec((BR2, CS), lambda j, r: (r, 0)),
            pl.BlockSpec((N, CR), lambda j, r: (0, 0)),
            pl.BlockSpec((N, OUT_C), lambda j, r: (0, 0)),
            pl.BlockSpec((N, OUT_C), lambda j, r: (0, 0)),
            pl.BlockSpec((8, OUT_C), lambda j, r: (0, 0)),
        ],
        out_specs=pl.BlockSpec(
            (BR2, OUT_C), lambda j, r: (jnp.where(j == K - 2, r, 0), 0)
        ),
        out_shape=jax.ShapeDtypeStruct((N, OUT_C), jnp.float32),
        scratch_shapes=[pltpu.VMEM((2, N, OUT_C), F8), pltpu.SemaphoreType.DMA],
    )(a8s, a8r, w, s1, c)
    return z


# speedup vs baseline: 1.3071x; 1.0221x over previous
import jax
import jax.numpy as jnp
from jax.experimental import pallas as pl
from jax.experimental.pallas import tpu as pltpu

N = 10000
IN_C = 512
HID = 256
OUT_C = 16
K = 10
ALPHA = 0.1

ADJ_SCALE = 16384.0
BR = 400
NB = N // BR
BR2 = 1000
NB2 = N // BR2
XBR = 1000
XNB = N // XBR
CS = 8320
CR = N - CS
F8 = jnp.float8_e4m3fn


def _mlp_kernel(x_ref, w1_ref, b1_ref, w2_ref, b2_ref, out_ref):
    h = jnp.dot(x_ref[...], w1_ref[...], preferred_element_type=jnp.float32)
    h = jnp.maximum(h + b1_ref[...], 0.0)
    out_ref[...] = (
        jnp.dot(h, w2_ref[...], preferred_element_type=jnp.float32)
        + b2_ref[...]
    )


def _quant_step0_kernel(
    adj_ref, x2_ref, a8s_ref, a8r_ref, s1_ref, w_ref, c_ref, s_ref
):
    r = pl.program_id(0)

    @pl.when(r == 0)
    def _():
        c0 = jnp.mean(x2_ref[...], axis=0, keepdims=True)
        c_ref[...] = jnp.broadcast_to(c0, (8, OUT_C))
        s_ref[:, :OUT_C] = (x2_ref[...] - c0).astype(F8)
        s_ref[:, OUT_C:] = jnp.ones((N, OUT_C), F8)

    q = (adj_ref[...] * ADJ_SCALE).astype(F8)
    a8s_ref[...] = q[:, :CS]
    a8r_ref[...] = q[:, CS:]
    d = jnp.dot(q, s_ref[...], preferred_element_type=jnp.float32)
    c = c_ref[0:1, :]
    rc = (d[:, OUT_C:] * (1.0 / ADJ_SCALE)) * c
    w = (1.0 - ALPHA) * rc + ALPHA * x2_ref[pl.ds(r * BR, BR), :]
    w_ref[...] = w
    z1 = ((1.0 - ALPHA) / ADJ_SCALE) * d[:, :OUT_C] + w
    s1_ref[...] = (z1 - c).astype(F8)


def _prop9_kernel(a8s_ref, a8r_ref, x2w_ref, s1_ref, c_ref, out_ref, s_ref, sem):
    j = pl.program_id(0)
    r = pl.program_id(1)

    @pl.when(jnp.logical_and(j == 0, r == 0))
    def _():
        cp = pltpu.make_async_copy(s1_ref, s_ref.at[0], sem)
        cp.start()
        cp.wait()

    d = jnp.dot(
        a8s_ref[...], s_ref[j % 2, :CS, :], preferred_element_type=jnp.float32
    ) + jnp.dot(
        a8r_ref[pl.ds(r * BR2, BR2), :],
        s_ref[j % 2, CS:, :],
        preferred_element_type=jnp.float32,
    )
    y = ((1.0 - ALPHA) / ADJ_SCALE) * d + x2w_ref[pl.ds(r * BR2, BR2), :]
    s_ref[(j + 1) % 2, pl.ds(r * BR2, BR2), :] = (y - c_ref[0:1, :]).astype(F8)

    @pl.when(j == K - 2)
    def _():
        out_ref[...] = y


def kernel(x, adj, W1, b1, W2, b2):
    b1r = b1.reshape(1, HID)
    b2r = b2.reshape(1, OUT_C)

    x2 = pl.pallas_call(
        _mlp_kernel,
        grid=(XNB,),
        in_specs=[
            pl.BlockSpec((XBR, IN_C), lambda i: (i, 0)),
            pl.BlockSpec((IN_C, HID), lambda i: (0, 0)),
            pl.BlockSpec((1, HID), lambda i: (0, 0)),
            pl.BlockSpec((HID, OUT_C), lambda i: (0, 0)),
            pl.BlockSpec((1, OUT_C), lambda i: (0, 0)),
        ],
        out_specs=pl.BlockSpec((XBR, OUT_C), lambda i: (i, 0)),
        out_shape=jax.ShapeDtypeStruct((N, OUT_C), jnp.float32),
    )(x, W1, b1r, W2, b2r)

    a8s, a8r, s1, w, c = pl.pallas_call(
        _quant_step0_kernel,
        grid=(NB,),
        in_specs=[
            pl.BlockSpec((BR, N), lambda r: (r, 0)),
            pl.BlockSpec((N, OUT_C), lambda r: (0, 0)),
        ],
        out_specs=[
            pl.BlockSpec((BR, CS), lambda r: (r, 0)),
            pl.BlockSpec((BR, CR), lambda r: (r, 0)),
            pl.BlockSpec((BR, OUT_C), lambda r: (r, 0)),
            pl.BlockSpec((BR, OUT_C), lambda r: (r, 0)),
            pl.BlockSpec((8, OUT_C), lambda r: (0, 0)),
        ],
        out_shape=[
            jax.ShapeDtypeStruct((N, CS), F8),
            jax.ShapeDtypeStruct((N, CR), F8),
            jax.ShapeDtypeStruct((N, OUT_C), F8),
            jax.ShapeDtypeStruct((N, OUT_C), jnp.float32),
            jax.ShapeDtypeStruct((8, OUT_C), jnp.float32),
        ],
        scratch_shapes=[pltpu.VMEM((N, 2 * OUT_C), F8)],
    )(adj, x2)

    z = pl.pallas_call(
        _prop9_kernel,
        grid=(K - 1, NB2),
        in_specs=[
            pl.BlockSpec((BR2, CS), lambda j, r: (r, 0)),
            pl.BlockSpec((N, CR), lambda j, r: (0, 0)),
            pl.BlockSpec((N, OUT_C), lambda j, r: (0, 0)),
            pl.BlockSpec((N, OUT_C), lambda j, r: (0, 0)),
            pl.BlockSpec((8, OUT_C), lambda j, r: (0, 0)),
        ],
        out_specs=pl.BlockSpec(
            (BR2, OUT_C), lambda j, r: (jnp.where(j == K - 2, r, 0), 0)
        ),
        out_shape=jax.ShapeDtypeStruct((N, OUT_C), jnp.float32),
        scratch_shapes=[pltpu.VMEM((2, N, OUT_C), F8), pltpu.SemaphoreType.DMA],
    )(a8s, a8r, w, s1, c)
    return z
